# Initial kernel scaffold; baseline (speedup 1.0000x reference)
import jax
import jax.numpy as jnp
from jax.experimental import pallas as pl


def kernel(x, edge_index, W1_self, W1_neigh, b1, W2_self, W2_neigh, b2, Wc, bc):
    raise NotImplementedError("write your pallas kernel here")


# speedup vs baseline: 4.7081x; 4.7081x over previous
import functools

import jax
import jax.numpy as jnp
from jax import lax
from jax.experimental import pallas as pl
from jax.experimental.pallas import tpu as pltpu
from jax.experimental.pallas import tpu_sc as plsc

N_NODES = 10000
N_PAD = 10240
N_EDGES = 320000
D = 128

NC = 2
NS = 16
NW = NC * NS
EPW = N_EDGES // NW
CHUNK = 80
NCHUNK = EPW // CHUNK
RPT = N_PAD // NS
ZR = 64

_mesh = plsc.VectorSubcoreMesh(core_axis_name="c", subcore_axis_name="s")


def _agg_body(y_hbm, src_hbm, dst_hbm, agg_out, sidx, didx, rows, zbuf,
              acc, sem):
  cc = lax.axis_index("c")
  ss = lax.axis_index("s")
  wid = ss * NC + cc

  zero16 = jnp.zeros((16,), jnp.float32)

  @pl.loop(0, ZR)
  def _zero_fill(r):
    @pl.loop(0, D // 16)
    def _(j):
      zbuf[r, pl.ds(j * 16, 16)] = zero16

  @pl.loop(0, RPT // ZR)
  def _zero_acc(r):
    pltpu.sync_copy(zbuf, acc.at[pl.ds(ss * RPT + r * ZR, ZR)])

  plsc.subcore_barrier()

  @pl.loop(0, NCHUNK)
  def _edges(i):
    base = wid * EPW + i * CHUNK
    pltpu.sync_copy(src_hbm.at[pl.ds(base, CHUNK)], sidx)
    pltpu.sync_copy(dst_hbm.at[pl.ds(base, CHUNK)], didx)
    pltpu.async_copy(y_hbm.at[sidx], rows, sem).wait()
    pltpu.sync_copy(rows, acc.at[didx], add=True)

  plsc.subcore_barrier()

  pltpu.sync_copy(acc.at[pl.ds(ss * RPT, RPT)],
                  agg_out.at[cc, pl.ds(ss * RPT, RPT)])


_agg = pl.kernel(
    _agg_body,
    out_type=[jax.ShapeDtypeStruct((NC, N_PAD, D), jnp.float32)],
    mesh=_mesh,
    scratch_types=[
        pltpu.VMEM((CHUNK,), jnp.int32),
        pltpu.VMEM((CHUNK,), jnp.int32),
        pltpu.VMEM((CHUNK, D), jnp.float32),
        pltpu.VMEM((ZR, D), jnp.float32),
        pltpu.VMEM_SHARED((N_PAD, D), jnp.float32),
        pltpu.SemaphoreType.DMA,
    ],
)


def _deg_body(dst_hbm, deg_out, didx, ones, zbuf, dacc):
  cc = lax.axis_index("c")
  ss = lax.axis_index("s")
  wid = ss * NC + cc

  zero16 = jnp.zeros((16,), jnp.float32)
  one16 = jnp.ones((16,), jnp.float32)

  @pl.loop(0, ZR)
  def _zero_fill(r):
    @pl.loop(0, D // 16)
    def _(j):
      zbuf[r, pl.ds(j * 16, 16)] = zero16

  @pl.loop(0, CHUNK)
  def _ones_fill(r):
    @pl.loop(0, D // 16)
    def _(j):
      ones[r, pl.ds(j * 16, 16)] = one16

  @pl.loop(0, RPT // ZR)
  def _zero_acc(r):
    pltpu.sync_copy(zbuf, dacc.at[pl.ds(ss * RPT + r * ZR, ZR)])

  plsc.subcore_barrier()

  @pl.loop(0, NCHUNK)
  def _edges(i):
    base = wid * EPW + i * CHUNK
    pltpu.sync_copy(dst_hbm.at[pl.ds(base, CHUNK)], didx)
    pltpu.sync_copy(ones, dacc.at[didx], add=True)

  plsc.subcore_barrier()

  pltpu.sync_copy(dacc.at[pl.ds(ss * RPT, RPT)],
                  deg_out.at[cc, pl.ds(ss * RPT, RPT)])


_deg = pl.kernel(
    _deg_body,
    out_type=[jax.ShapeDtypeStruct((NC, N_PAD, D), jnp.float32)],
    mesh=_mesh,
    scratch_types=[
        pltpu.VMEM((CHUNK,), jnp.int32),
        pltpu.VMEM((CHUNK, D), jnp.float32),
        pltpu.VMEM((ZR, D), jnp.float32),
        pltpu.VMEM_SHARED((N_PAD, D), jnp.float32),
    ],
)


_R = 2000
_GRID = N_NODES // _R


def _dot(a, b):
  return jnp.dot(a, b, preferred_element_type=jnp.float32,
                 precision=lax.Precision.HIGHEST)


def _pre_body(x_ref, ws_ref, wn_ref, b_ref, z_ref, y_ref):
  xb = x_ref[...]
  z_ref[...] = _dot(xb, ws_ref[...]) + b_ref[...]
  y_ref[...] = _dot(xb, wn_ref[...])


def _mix_body(z_ref, agg_ref, dg_ref, ws_ref, wn_ref, b_ref, z2_ref, y2_ref):
  a = agg_ref[0] + agg_ref[1]
  d = dg_ref[0, :, 0:1] + dg_ref[1, :, 0:1]
  inv = 1.0 / jnp.maximum(d, 1.0)
  h = jnp.maximum(z_ref[...] + a * inv, 0.0)
  z2_ref[...] = _dot(h, ws_ref[...]) + b_ref[...]
  y2_ref[...] = _dot(h, wn_ref[...])


def _fin_body(z_ref, agg_ref, dg_ref, wc_ref, bc_ref, o_ref):
  a = agg_ref[0] + agg_ref[1]
  d = dg_ref[0, :, 0:1] + dg_ref[1, :, 0:1]
  inv = 1.0 / jnp.maximum(d, 1.0)
  h = jnp.maximum(z_ref[...] + a * inv, 0.0)
  o_ref[...] = _dot(h, wc_ref[...]) + bc_ref[...]


def _row_spec(r, d):
  return pl.BlockSpec((r, d), lambda i: (i, 0))


def _part_spec(r, d):
  return pl.BlockSpec((NC, r, d), lambda i: (0, i, 0))


def _deg_spec(r):
  return pl.BlockSpec((NC, r, D), lambda i: (0, i, 0))


def _full_spec(a, b):
  return pl.BlockSpec((a, b), lambda i: (0, 0))


def _pre_call(x, ws, wn, b):
  return pl.pallas_call(
      _pre_body,
      grid=(_GRID,),
      in_specs=[_row_spec(_R, D), _full_spec(D, D), _full_spec(D, D),
                _full_spec(1, D)],
      out_specs=[_row_spec(_R, D), _row_spec(_R, D)],
      out_shape=[jax.ShapeDtypeStruct((N_NODES, D), jnp.float32),
                 jax.ShapeDtypeStruct((N_PAD, D), jnp.float32)],
  )(x, ws, wn, b)


def _mix_call(z, agg, dg, ws, wn, b):
  return pl.pallas_call(
      _mix_body,
      grid=(_GRID,),
      in_specs=[_row_spec(_R, D), _part_spec(_R, D), _deg_spec(_R),
                _full_spec(D, D), _full_spec(D, D), _full_spec(1, D)],
      out_specs=[_row_spec(_R, D), _row_spec(_R, D)],
      out_shape=[jax.ShapeDtypeStruct((N_NODES, D), jnp.float32),
                 jax.ShapeDtypeStruct((N_PAD, D), jnp.float32)],
  )(z, agg, dg, ws, wn, b)


def _fin_call(z, agg, dg, wc, bc):
  return pl.pallas_call(
      _fin_body,
      grid=(_GRID,),
      in_specs=[_row_spec(_R, D), _part_spec(_R, D), _deg_spec(_R),
                _full_spec(D, D), _full_spec(1, D)],
      out_specs=_row_spec(_R, D),
      out_shape=jax.ShapeDtypeStruct((N_NODES, D), jnp.float32),
  )(z, agg, dg, wc, bc)


@jax.jit
def kernel(x, edge_index, W1_self, W1_neigh, b1, W2_self, W2_neigh, b2, Wc,
           bc):
  ei = edge_index.astype(jnp.int32)
  src = ei[0]
  dst = ei[1]

  (dg,) = _deg(dst)
  z1, y1 = _pre_call(x, W1_self, W1_neigh, b1.reshape(1, D))
  (agg1,) = _agg(y1, src, dst)
  z2, y2 = _mix_call(z1, agg1, dg, W2_self, W2_neigh, b2.reshape(1, D))
  (agg2,) = _agg(y2, src, dst)

  wc_pad = jnp.zeros((D, D), jnp.float32).at[:, :40].set(Wc)
  bc_pad = jnp.zeros((1, D), jnp.float32).at[0, :40].set(bc)
  out = _fin_call(z2, agg2, dg, wc_pad, bc_pad)
  return out[:, :40]
